# baseline (speedup 1.0000x reference)
import functools

import jax
import jax.numpy as jnp
from jax import lax
from jax.experimental import pallas as pl
from jax.experimental.pallas import tpu as pltpu
from jax.experimental.pallas import tpu_sc as plsc

_GLOBAL_MEAN = 3.5
_MIN_R = 1.0
_MAX_R = 5.0


@functools.partial(jax.jit, static_argnums=(6, 7))
def _sc_gather(user_idx, item_idx, user_emb, item_emb, user_bias_flat,
               item_bias_flat, B, D):
    info = plsc.get_sparse_core_info()
    nw = info.num_cores * info.num_subcores
    nc = info.num_cores
    b_per_w = B // nw
    mesh = plsc.VectorSubcoreMesh(core_axis_name="c", subcore_axis_name="s")

    @functools.partial(
        pl.kernel,
        out_type=(
            jax.ShapeDtypeStruct((B, D), jnp.float32),
            jax.ShapeDtypeStruct((B, D), jnp.float32),
            jax.ShapeDtypeStruct((B,), jnp.float32),
            jax.ShapeDtypeStruct((B,), jnp.float32),
        ),
        mesh=mesh,
        compiler_params=pltpu.CompilerParams(use_tc_tiling_on_sc=False),
        scratch_types=[
            pltpu.VMEM((b_per_w,), jnp.int32),
            pltpu.VMEM((b_per_w,), jnp.int32),
            pltpu.VMEM((b_per_w, D), jnp.float32),
            pltpu.VMEM((b_per_w, D), jnp.float32),
            pltpu.VMEM((b_per_w,), jnp.float32),
            pltpu.VMEM((b_per_w,), jnp.float32),
            pltpu.SemaphoreType.DMA,
        ],
    )
    def gather_kernel(uidx_hbm, iidx_hbm, uemb_hbm, iemb_hbm, ub_hbm, ib_hbm,
                      uvec_out, ivec_out, ub_out, ib_out,
                      uidx_v, iidx_v, urows_v, irows_v, ub_v, ib_v, sem):
        wid = lax.axis_index("s") * nc + lax.axis_index("c")
        base = wid * b_per_w
        pltpu.sync_copy(uidx_hbm.at[pl.ds(base, b_per_w)], uidx_v)
        pltpu.sync_copy(iidx_hbm.at[pl.ds(base, b_per_w)], iidx_v)
        cu = pltpu.async_copy(uemb_hbm.at[uidx_v], urows_v, sem)
        ci = pltpu.async_copy(iemb_hbm.at[iidx_v], irows_v, sem)
        cub = pltpu.async_copy(ub_hbm.at[uidx_v], ub_v, sem)
        cib = pltpu.async_copy(ib_hbm.at[iidx_v], ib_v, sem)
        cu.wait()
        ci.wait()
        cub.wait()
        cib.wait()
        pltpu.sync_copy(urows_v, uvec_out.at[pl.ds(base, b_per_w)])
        pltpu.sync_copy(irows_v, ivec_out.at[pl.ds(base, b_per_w)])
        pltpu.sync_copy(ub_v, ub_out.at[pl.ds(base, b_per_w)])
        pltpu.sync_copy(ib_v, ib_out.at[pl.ds(base, b_per_w)])

    return gather_kernel(user_idx, item_idx, user_emb, item_emb,
                         user_bias_flat, item_bias_flat)


def _mlp_body(uvec_ref, ivec_ref, ub_ref, ib_ref, w1u_ref, w1i_ref, b1_ref,
              w2_ref, b2_ref, w3_ref, cst_ref, out_ref):
    h = (jnp.dot(uvec_ref[...], w1u_ref[...], preferred_element_type=jnp.float32)
         + jnp.dot(ivec_ref[...], w1i_ref[...], preferred_element_type=jnp.float32)
         + b1_ref[...])
    h = jnp.maximum(h, 0.0)
    h2 = jnp.dot(h, w2_ref[...], preferred_element_type=jnp.float32) + b2_ref[...]
    h2 = jnp.maximum(h2, 0.0)
    inter = jnp.sum(h2 * w3_ref[...], axis=1)
    pred = cst_ref[0, 0] + ub_ref[...] + ib_ref[...] + inter
    out_ref[...] = jnp.clip(pred, _MIN_R, _MAX_R)


@functools.partial(jax.jit, static_argnums=(11, 12, 13))
def _tc_mlp(uvec, ivec, ub, ib, w1u, w1i, b1r, w2t, b2r, w3r, cst, B, D, H):
    bm = 2048
    grid = (B // bm,)
    return pl.pallas_call(
        _mlp_body,
        grid=grid,
        in_specs=[
            pl.BlockSpec((bm, D), lambda i: (i, 0)),
            pl.BlockSpec((bm, D), lambda i: (i, 0)),
            pl.BlockSpec((bm,), lambda i: (i,)),
            pl.BlockSpec((bm,), lambda i: (i,)),
            pl.BlockSpec((D, H), lambda i: (0, 0)),
            pl.BlockSpec((D, H), lambda i: (0, 0)),
            pl.BlockSpec((1, H), lambda i: (0, 0)),
            pl.BlockSpec((H, 32), lambda i: (0, 0)),
            pl.BlockSpec((1, 32), lambda i: (0, 0)),
            pl.BlockSpec((1, 32), lambda i: (0, 0)),
            pl.BlockSpec((1, 1), lambda i: (0, 0)),
        ],
        out_specs=pl.BlockSpec((bm,), lambda i: (i,)),
        out_shape=jax.ShapeDtypeStruct((B,), jnp.float32),
    )(uvec, ivec, ub, ib, w1u, w1i, b1r, w2t, b2r, w3r, cst)


def kernel(user_idx, item_idx, user_emb, item_emb, user_bias, item_bias,
           W1, b1, W2, b2, W3, b3):
    B = user_idx.shape[0]
    U, D = user_emb.shape
    H = W1.shape[0]

    uvec, ivec, ub, ib = _sc_gather(
        user_idx, item_idx, user_emb, item_emb,
        user_bias.reshape(-1), item_bias.reshape(-1), B, D)

    w1u = W1[:, :D].T
    w1i = W1[:, D:].T
    b1r = b1.reshape(1, H)
    w2t = W2.T
    b2r = b2.reshape(1, 32)
    w3r = W3.reshape(1, 32)
    cst = (_GLOBAL_MEAN + b3).reshape(1, 1)

    return _tc_mlp(uvec, ivec, ub, ib, w1u, w1i, b1r, w2t, b2r, w3r, cst,
                   B, D, H)
